# Initial kernel scaffold; baseline (speedup 1.0000x reference)
import jax
import jax.numpy as jnp
from jax.experimental import pallas as pl


def kernel(x, emb_weight):
    raise NotImplementedError("write your pallas kernel here")


# speedup vs baseline: 1.6699x; 1.6699x over previous
import jax
import jax.numpy as jnp
from jax.experimental import pallas as pl
from jax.experimental.pallas import tpu as pltpu

B = 4
S = 4096
PE_DIM = 2048
BS = 512


def _add_kernel(x_ref, pe_ref, out_ref):
    out_ref[0, :, :] = x_ref[0, :, :] + pe_ref[:, :]


def kernel(x, emb_weight):
    b, s, d = x.shape
    grid = (s // BS, b)
    return pl.pallas_call(
        _add_kernel,
        grid=grid,
        in_specs=[
            pl.BlockSpec((1, BS, d), lambda i, j: (j, i, 0)),
            pl.BlockSpec((BS, d), lambda i, j: (i, 0)),
        ],
        out_specs=pl.BlockSpec((1, BS, d), lambda i, j: (j, i, 0)),
        out_shape=jax.ShapeDtypeStruct((b, s, d), x.dtype),
        compiler_params=pltpu.CompilerParams(
            dimension_semantics=("arbitrary", "arbitrary"),
        ),
    )(x, emb_weight)


# speedup vs baseline: 1.7429x; 1.0437x over previous
import jax
import jax.numpy as jnp
from jax.experimental import pallas as pl
from jax.experimental.pallas import tpu as pltpu

B = 4
S = 4096
PE_DIM = 2048
BS = 1024


def _add_kernel(x_ref, pe_ref, out_ref):
    out_ref[0, :, :] = x_ref[0, :, :] + pe_ref[:, :]


def kernel(x, emb_weight):
    b, s, d = x.shape
    grid = (s // BS, b)
    return pl.pallas_call(
        _add_kernel,
        grid=grid,
        in_specs=[
            pl.BlockSpec((1, BS, d), lambda i, j: (j, i, 0)),
            pl.BlockSpec((BS, d), lambda i, j: (i, 0)),
        ],
        out_specs=pl.BlockSpec((1, BS, d), lambda i, j: (j, i, 0)),
        out_shape=jax.ShapeDtypeStruct((b, s, d), x.dtype),
        compiler_params=pltpu.CompilerParams(
            dimension_semantics=("arbitrary", "arbitrary"),
        ),
    )(x, emb_weight)
